# baseline (speedup 1.0000x reference)
import jax
import jax.numpy as jnp
from jax.experimental import pallas as pl
from jax.experimental.pallas import tpu as pltpu

G = 16
RPI = 24
M = G * RPI
SLOP = 8
BF = jnp.bfloat16


def _row_mask(like):
    r = jax.lax.broadcasted_iota(jnp.int32, (like.shape[0], 1), 0) % RPI
    keep = (r >= 4) & (r < 20)
    return jnp.where(keep, like, 0.0)


def _conv_class(slices, band_ref, bias_ref):
    acc = jnp.dot(slices[0], band_ref[0], preferred_element_type=jnp.float32)
    for dy in range(1, 5):
        acc = acc + jnp.dot(slices[dy], band_ref[dy],
                            preferred_element_type=jnp.float32)
    return jnp.maximum(acc + bias_ref[...], 0.0)


def _conv_net_kernel(x_ref, b1_ref, b2_ref, b3_ref, a1_ref, a2_ref, a3_ref,
                     o_ref, s2a_ref, s2b_ref, s3_ref):
    def x_slice(c, dy):
        v = c + dy - 2
        s = v % 4
        off = (v - s) // 4
        return x_ref[s, 0, 1 + off:1 + off + M, :]

    for r, s2_ref in ((0, s2a_ref), (1, s2b_ref)):
        ya = _conv_class([x_slice(2 * r, dy) for dy in range(5)],
                         b1_ref, a1_ref)
        yb = _conv_class([x_slice(2 * r + 1, dy) for dy in range(5)],
                         b1_ref, a1_ref)
        p = jnp.maximum(ya, yb)
        p = jnp.maximum(p[:, 0:256], p[:, 256:512])
        s2_ref[1:1 + M, :] = _row_mask(p).astype(BF)

    def s2_slice(c, dy):
        v = c + dy - 2
        s = v % 2
        off = (v - s) // 2
        ref = s2a_ref if s == 0 else s2b_ref
        return ref[1 + off:1 + off + M, :]

    y2 = [None] * 2
    for c in range(2):
        y2[c] = _conv_class([s2_slice(c, dy) for dy in range(5)],
                            b2_ref, a2_ref)

    p2 = jnp.maximum(y2[0], y2[1])
    p2 = jnp.maximum(p2[:, 0:256], p2[:, 256:512])
    s3_ref[2:2 + M, :] = _row_mask(p2).astype(BF)

    y3 = _conv_class([s3_ref[dy:dy + M, :] for dy in range(5)],
                     b3_ref, a3_ref)
    y3 = y3.astype(BF)

    for g in range(G):
        o_ref[:, 0, g, :] = y3[g * RPI + 4:g * RPI + 20, :]


def _fc_kernel(f_ref, w_ref, b1_ref, w2_ref, b2_ref, o_ref, acc_ref):
    j = pl.program_id(0)

    @pl.when(j == 0)
    def _():
        acc_ref[...] = jnp.zeros_like(acc_ref)

    w = w_ref[...].reshape(1920, 128)
    acc_ref[...] += jnp.dot(f_ref[0], w, preferred_element_type=jnp.float32)

    @pl.when(j == pl.num_programs(0) - 1)
    def _():
        h = acc_ref[...] + b1_ref[...]
        o_ref[...] = jnp.dot(h, w2_ref[...],
                             preferred_element_type=jnp.float32) + b2_ref[...]


def _make_bands(w, cout, cin, width, split_x=True):
    f32 = jnp.float32
    w4 = w.reshape(cout, 5, 5, cin).astype(f32)
    eyes = jnp.stack([jnp.eye(width, width, 2 - e, dtype=f32)
                      for e in range(5)])
    band = jnp.einsum('odec,eix->dciox', w4, eyes)
    band = band.reshape(5, cin * width, cout, width)
    if split_x:
        band = band.reshape(5, cin * width, cout, width // 2, 2)
        band = band.transpose(0, 1, 4, 2, 3)
    return band.reshape(5, cin * width, cout * width).astype(BF)


def kernel(x, c1w, c1b, c2w, c2b, c3w, c3b, fc1_wt, fc1_b, fc2_wt, fc2_b):
    f32 = jnp.float32
    B = x.shape[0]
    ngrp = B // G

    x4 = x.astype(BF).reshape(B, 16, 4, 64).transpose(2, 0, 1, 3)
    x4 = jnp.pad(x4, ((0, 0), (0, 0), (5, 3), (0, 0)))
    x4 = x4.reshape(4, ngrp, G * RPI, 64)
    x4 = jnp.pad(x4, ((0, 0), (0, 0), (0, SLOP), (0, 0)))

    band1 = _make_bands(c1w, 8, 1, 64)
    band2 = _make_bands(c2w, 16, 8, 32)
    band3 = _make_bands(c3w, 120, 16, 16, split_x=False)

    def tile_bias(b, width, split_x=True):
        t = jnp.repeat(b.reshape(-1), width // (2 if split_x else 1))
        if split_x:
            t = jnp.tile(t, (2,))
        return t.reshape(1, -1).astype(f32)

    a1 = tile_bias(c1b, 64)
    a2 = tile_bias(c2b, 32)
    a3 = tile_bias(c3b, 16, split_x=False)

    feat = pl.pallas_call(
        _conv_net_kernel,
        out_shape=jax.ShapeDtypeStruct((16, B // G, G, 1920), BF),
        grid_spec=pltpu.PrefetchScalarGridSpec(
            num_scalar_prefetch=0,
            grid=(ngrp,),
            in_specs=[
                pl.BlockSpec((4, 1, G * RPI + SLOP, 64),
                             lambda i: (0, i, 0, 0)),
                pl.BlockSpec(band1.shape, lambda i: (0, 0, 0)),
                pl.BlockSpec(band2.shape, lambda i: (0, 0, 0)),
                pl.BlockSpec(band3.shape, lambda i: (0, 0, 0)),
                pl.BlockSpec(a1.shape, lambda i: (0, 0)),
                pl.BlockSpec(a2.shape, lambda i: (0, 0)),
                pl.BlockSpec(a3.shape, lambda i: (0, 0)),
            ],
            out_specs=pl.BlockSpec((16, 1, G, 1920), lambda i: (0, i, 0, 0)),
            scratch_shapes=[
                pltpu.VMEM((G * RPI + SLOP, 256), BF),
                pltpu.VMEM((G * RPI + SLOP, 256), BF),
                pltpu.VMEM((G * RPI + SLOP, 256), BF),
            ],
        ),
        compiler_params=pltpu.CompilerParams(
            dimension_semantics=("parallel",)),
    )(x4, band1, band2, band3, a1, a2, a3)

    f2 = feat.reshape(16, B, 1920)
    w3 = fc1_wt.reshape(120, 256, 128).astype(BF)

    out = pl.pallas_call(
        _fc_kernel,
        out_shape=jax.ShapeDtypeStruct((B, 7), f32),
        grid_spec=pltpu.PrefetchScalarGridSpec(
            num_scalar_prefetch=0,
            grid=(16,),
            in_specs=[
                pl.BlockSpec((1, B, 1920), lambda j: (j, 0, 0)),
                pl.BlockSpec((120, 16, 128), lambda j: (0, j, 0)),
                pl.BlockSpec((1, 128), lambda j: (0, 0)),
                pl.BlockSpec((128, 7), lambda j: (0, 0)),
                pl.BlockSpec((1, 7), lambda j: (0, 0)),
            ],
            out_specs=pl.BlockSpec((B, 7), lambda j: (0, 0)),
            scratch_shapes=[pltpu.VMEM((B, 128), f32)],
        ),
        compiler_params=pltpu.CompilerParams(
            dimension_semantics=("arbitrary",)),
    )(f2, w3, fc1_b, fc2_wt, fc2_b)
    return out


# speedup vs baseline: 1.0320x; 1.0320x over previous
import jax
import jax.numpy as jnp
from jax.experimental import pallas as pl
from jax.experimental.pallas import tpu as pltpu

G = 16
RPI = 24
M = G * RPI
SLOP = 8
BF = jnp.bfloat16


def _row_mask(like):
    r = jax.lax.broadcasted_iota(jnp.int32, (like.shape[0], 1), 0) % RPI
    keep = (r >= 4) & (r < 20)
    return jnp.where(keep, like, 0.0)


def _conv_class(slices, band_ref, bias_ref):
    acc = jnp.dot(slices[0], band_ref[0], preferred_element_type=jnp.float32)
    for dy in range(1, 5):
        acc = acc + jnp.dot(slices[dy], band_ref[dy],
                            preferred_element_type=jnp.float32)
    return jnp.maximum(acc + bias_ref[...], 0.0)


def _conv_net_kernel(x_ref, b1_ref, b2_ref, b3_ref, a1_ref, a2_ref, a3_ref,
                     o_ref, s2a_ref, s2b_ref, s3_ref):
    def x_slice(c, dy):
        v = c + dy - 2
        s = v % 4
        off = (v - s) // 4
        return x_ref[s, 0, 1 + off:1 + off + M, :]

    for r, s2_ref in ((0, s2a_ref), (1, s2b_ref)):
        ya = _conv_class([x_slice(2 * r, dy) for dy in range(5)],
                         b1_ref, a1_ref)
        yb = _conv_class([x_slice(2 * r + 1, dy) for dy in range(5)],
                         b1_ref, a1_ref)
        p = jnp.maximum(ya, yb)
        p = jnp.maximum(p[:, 0:256], p[:, 256:512])
        s2_ref[1:1 + M, :] = _row_mask(p).astype(BF)

    def s2_slice(c, dy):
        v = c + dy - 2
        s = v % 2
        off = (v - s) // 2
        ref = s2a_ref if s == 0 else s2b_ref
        return ref[1 + off:1 + off + M, :]

    y2 = [None] * 2
    for c in range(2):
        y2[c] = _conv_class([s2_slice(c, dy) for dy in range(5)],
                            b2_ref, a2_ref)

    p2 = jnp.maximum(y2[0], y2[1])
    p2 = jnp.maximum(p2[:, 0:256], p2[:, 256:512])
    s3_ref[2:2 + M, :] = _row_mask(p2).astype(BF)

    y3 = _conv_class([s3_ref[dy:dy + M, :] for dy in range(5)],
                     b3_ref, a3_ref)
    y3 = y3.astype(BF)

    for g in range(G):
        o_ref[:, 0, g, :] = y3[g * RPI + 4:g * RPI + 20, :]


def _fc_kernel(f_ref, w_ref, b1_ref, w2_ref, b2_ref, o_ref, acc_ref):
    j = pl.program_id(0)

    @pl.when(j == 0)
    def _():
        acc_ref[...] = jnp.zeros_like(acc_ref)

    w = w_ref[...].reshape(1920, 128).astype(BF)
    acc_ref[...] += jnp.dot(f_ref[0], w, preferred_element_type=jnp.float32)

    @pl.when(j == pl.num_programs(0) - 1)
    def _():
        h = acc_ref[...] + b1_ref[...]
        o_ref[...] = jnp.dot(h, w2_ref[...],
                             preferred_element_type=jnp.float32) + b2_ref[...]


def _make_bands(w, cout, cin, width, split_x=True):
    f32 = jnp.float32
    w4 = w.reshape(cout, 5, 5, cin).astype(f32)
    eyes = jnp.stack([jnp.eye(width, width, 2 - e, dtype=f32)
                      for e in range(5)])
    band = jnp.einsum('odec,eix->dciox', w4, eyes)
    band = band.reshape(5, cin * width, cout, width)
    if split_x:
        band = band.reshape(5, cin * width, cout, width // 2, 2)
        band = band.transpose(0, 1, 4, 2, 3)
    return band.reshape(5, cin * width, cout * width).astype(BF)


def kernel(x, c1w, c1b, c2w, c2b, c3w, c3b, fc1_wt, fc1_b, fc2_wt, fc2_b):
    f32 = jnp.float32
    B = x.shape[0]
    ngrp = B // G

    x4 = x.astype(BF).reshape(B, 16, 4, 64).transpose(2, 0, 1, 3)
    x4 = jnp.pad(x4, ((0, 0), (0, 0), (5, 3), (0, 0)))
    x4 = x4.reshape(4, ngrp, G * RPI, 64)
    x4 = jnp.pad(x4, ((0, 0), (0, 0), (0, SLOP), (0, 0)))

    band1 = _make_bands(c1w, 8, 1, 64)
    band2 = _make_bands(c2w, 16, 8, 32)
    band3 = _make_bands(c3w, 120, 16, 16, split_x=False)

    def tile_bias(b, width, split_x=True):
        t = jnp.repeat(b.reshape(-1), width // (2 if split_x else 1))
        if split_x:
            t = jnp.tile(t, (2,))
        return t.reshape(1, -1).astype(f32)

    a1 = tile_bias(c1b, 64)
    a2 = tile_bias(c2b, 32)
    a3 = tile_bias(c3b, 16, split_x=False)

    feat = pl.pallas_call(
        _conv_net_kernel,
        out_shape=jax.ShapeDtypeStruct((16, B // G, G, 1920), BF),
        grid_spec=pltpu.PrefetchScalarGridSpec(
            num_scalar_prefetch=0,
            grid=(ngrp,),
            in_specs=[
                pl.BlockSpec((4, 1, G * RPI + SLOP, 64),
                             lambda i: (0, i, 0, 0)),
                pl.BlockSpec(band1.shape, lambda i: (0, 0, 0)),
                pl.BlockSpec(band2.shape, lambda i: (0, 0, 0)),
                pl.BlockSpec(band3.shape, lambda i: (0, 0, 0)),
                pl.BlockSpec(a1.shape, lambda i: (0, 0)),
                pl.BlockSpec(a2.shape, lambda i: (0, 0)),
                pl.BlockSpec(a3.shape, lambda i: (0, 0)),
            ],
            out_specs=pl.BlockSpec((16, 1, G, 1920), lambda i: (0, i, 0, 0)),
            scratch_shapes=[
                pltpu.VMEM((G * RPI + SLOP, 256), BF),
                pltpu.VMEM((G * RPI + SLOP, 256), BF),
                pltpu.VMEM((G * RPI + SLOP, 256), BF),
            ],
        ),
        compiler_params=pltpu.CompilerParams(
            dimension_semantics=("parallel",)),
    )(x4, band1, band2, band3, a1, a2, a3)

    f2 = feat.reshape(16, B, 1920)
    w3 = fc1_wt.reshape(120, 256, 128)

    out = pl.pallas_call(
        _fc_kernel,
        out_shape=jax.ShapeDtypeStruct((B, 7), f32),
        grid_spec=pltpu.PrefetchScalarGridSpec(
            num_scalar_prefetch=0,
            grid=(16,),
            in_specs=[
                pl.BlockSpec((1, B, 1920), lambda j: (j, 0, 0)),
                pl.BlockSpec((120, 16, 128), lambda j: (0, j, 0)),
                pl.BlockSpec((1, 128), lambda j: (0, 0)),
                pl.BlockSpec((128, 7), lambda j: (0, 0)),
                pl.BlockSpec((1, 7), lambda j: (0, 0)),
            ],
            out_specs=pl.BlockSpec((B, 7), lambda j: (0, 0)),
            scratch_shapes=[pltpu.VMEM((B, 128), f32)],
        ),
        compiler_params=pltpu.CompilerParams(
            dimension_semantics=("arbitrary",)),
    )(f2, w3, fc1_b, fc2_wt, fc2_b)
    return out
